# baseline (speedup 1.0000x reference)
import functools

import jax
import jax.numpy as jnp
from jax import lax
from jax.experimental import pallas as pl
from jax.experimental.pallas import tpu as pltpu
from jax.experimental.pallas import tpu_sc as plsc

B = 16384
D = 32
V = 1_000_001
R16 = (D * V) // 16
CHUNK = 128
_info = plsc.get_sparse_core_info()
NC, NS = _info.num_cores, _info.num_subcores
NW = NC * NS
BPW = B // NW
KPW = BPW // CHUNK
LANES = 16
DL = D * LANES


@functools.partial(
    pl.kernel,
    mesh=plsc.VectorSubcoreMesh(core_axis_name="c", subcore_axis_name="s"),
    compiler_params=pltpu.CompilerParams(use_tc_tiling_on_sc=False),
    out_type=[
        jax.ShapeDtypeStruct((B, DL), jnp.float32),
        jax.ShapeDtypeStruct((B, DL), jnp.float32),
    ],
    scratch_types=[
        pltpu.VMEM((BPW,), jnp.int32),
        pltpu.VMEM((BPW,), jnp.int32),
        pltpu.VMEM((2, KPW, CHUNK), jnp.int32),
        pltpu.VMEM((2, KPW, CHUNK), jnp.int32),
        pltpu.VMEM((BPW, LANES), jnp.float32),
        pltpu.VMEM((BPW, LANES), jnp.float32),
        pltpu.VMEM((BPW, LANES), jnp.float32),
        pltpu.VMEM((BPW, LANES), jnp.float32),
        pltpu.SemaphoreType.DMA,
        pltpu.SemaphoreType.DMA,
    ],
)
def _sc_gather(uid_hbm, iid_hbm, utab_hbm, itab_hbm, uout_hbm, iout_hbm,
               uids_v, iids_v, uridx_v, iridx_v,
               ustage0_v, ustage1_v, istage0_v, istage1_v, sem_u, sem_i):
    ustage = (ustage0_v, ustage1_v)
    istage = (istage0_v, istage1_v)
    wid = lax.axis_index("s") * NC + lax.axis_index("c")
    b0 = wid * BPW
    pltpu.sync_copy(uid_hbm.at[pl.ds(b0, BPW)], uids_v)
    pltpu.sync_copy(iid_hbm.at[pl.ds(b0, BPW)], iids_v)

    def build_and_fire(d, par):
        for ids_v, ridx_v in ((uids_v, uridx_v), (iids_v, iridx_v)):
            for j in range(BPW // LANES):
                k, c = divmod(j * LANES, CHUNK)
                e = ids_v[pl.ds(j * LANES, LANES)] + d * V
                ridx_v[par, k, pl.ds(c, LANES)] = lax.shift_right_logical(
                    e, 4)
        for k in range(KPW):
            sl = pl.ds(k * CHUNK, CHUNK)
            pltpu.async_copy(utab_hbm.at[uridx_v.at[par, k]],
                             ustage[par].at[sl, :], sem_u)
            pltpu.async_copy(itab_hbm.at[iridx_v.at[par, k]],
                             istage[par].at[sl, :], sem_i)

    def drain(par):
        pltpu.make_async_copy(utab_hbm.at[uridx_v.at[par, 0]],
                              ustage[par], sem_u).wait()
        pltpu.make_async_copy(itab_hbm.at[iridx_v.at[par, 0]],
                              istage[par], sem_i).wait()

    def flush(d, par):
        dst = pl.ds(b0, BPW)
        pltpu.sync_copy(ustage[par],
                        uout_hbm.at[dst, pl.ds(d * LANES, LANES)])
        pltpu.sync_copy(istage[par],
                        iout_hbm.at[dst, pl.ds(d * LANES, LANES)])

    build_and_fire(0, 0)

    def body(t, carry):
        d0 = 2 * t
        build_and_fire(d0 + 1, 1)
        drain(0)
        flush(d0, 0)
        build_and_fire(d0 + 2, 0)
        drain(1)
        flush(d0 + 1, 1)
        return carry

    lax.fori_loop(0, D // 2 - 1, body, 0)
    build_and_fire(D - 1, 1)
    drain(0)
    flush(D - 2, 0)
    drain(1)
    flush(D - 1, 1)


BLK = 1024


def _mlp_body(ustg_ref, istg_ref, uid_ref, iid_ref, w1u_ref, w1i_ref,
              b1_ref, w2_ref, b2_ref, w3_ref, b3_ref, o_ref):
    p = lax.broadcasted_iota(jnp.int32, (1, DL), 1)
    dp = lax.shift_right_logical(p, 4)
    tp = lax.bitwise_and(p, 15)

    def masked(stg_ref, id_ref):
        lane = lax.bitwise_and(dp * V + id_ref[...], 15)
        return jnp.where(lane == tp, stg_ref[...], 0.0)

    h = jnp.dot(masked(ustg_ref, uid_ref), w1u_ref[...],
                preferred_element_type=jnp.float32)
    h += jnp.dot(masked(istg_ref, iid_ref), w1i_ref[...],
                 preferred_element_type=jnp.float32)
    h = jnp.maximum(h + b1_ref[...], 0.0)
    h = jnp.maximum(jnp.dot(h, w2_ref[...],
                            preferred_element_type=jnp.float32)
                    + b2_ref[...], 0.0)
    o = jnp.sum(h * w3_ref[...], axis=1, keepdims=True) + b3_ref[...]
    o_ref[...] = o


def _mlp(ustg, istg, uid2, iid2, w1u, w1i, b1, w2, b2, w3t, b3):
    return pl.pallas_call(
        _mlp_body,
        grid=(B // BLK,),
        in_specs=[
            pl.BlockSpec((BLK, DL), lambda i: (i, 0)),
            pl.BlockSpec((BLK, DL), lambda i: (i, 0)),
            pl.BlockSpec((BLK, 1), lambda i: (i, 0)),
            pl.BlockSpec((BLK, 1), lambda i: (i, 0)),
            pl.BlockSpec((DL, 256), lambda i: (0, 0)),
            pl.BlockSpec((DL, 256), lambda i: (0, 0)),
            pl.BlockSpec((1, 256), lambda i: (0, 0)),
            pl.BlockSpec((256, 64), lambda i: (0, 0)),
            pl.BlockSpec((1, 64), lambda i: (0, 0)),
            pl.BlockSpec((1, 64), lambda i: (0, 0)),
            pl.BlockSpec((1, 1), lambda i: (0, 0)),
        ],
        out_specs=pl.BlockSpec((BLK, 1), lambda i: (i, 0)),
        out_shape=jax.ShapeDtypeStruct((B, 1), jnp.float32),
    )(ustg, istg, uid2, iid2, w1u, w1i, b1, w2, b2, w3t, b3)


def kernel(user_id, item_id, user_table, item_table, W1, b1, W2, b2, W3, b3):
    uid = user_id.astype(jnp.int32)
    iid = item_id.astype(jnp.int32)
    urows16 = user_table.T.reshape(R16, LANES)
    irows16 = item_table.T.reshape(R16, LANES)
    ustg, istg = _sc_gather(uid, iid, urows16, irows16)
    w1uexp = jnp.repeat(W1[:D, :], LANES, axis=0)
    w1iexp = jnp.repeat(W1[D:, :], LANES, axis=0)
    return _mlp(ustg, istg, uid.reshape(B, 1), iid.reshape(B, 1),
                w1uexp, w1iexp, b1.reshape(1, 256),
                W2, b2.reshape(1, 64), W3.reshape(1, 64), b3.reshape(1, 1))


# speedup vs baseline: 1.0038x; 1.0038x over previous
import functools

import jax
import jax.numpy as jnp
from jax import lax
from jax.experimental import pallas as pl
from jax.experimental.pallas import tpu as pltpu
from jax.experimental.pallas import tpu_sc as plsc

B = 16384
D = 32
V = 1_000_001
R16 = (D * V) // 16
CHUNK = 128
_info = plsc.get_sparse_core_info()
NC, NS = _info.num_cores, _info.num_subcores
NW = NC * NS
BPW = B // NW
KPW = BPW // CHUNK
LANES = 16
DL = D * LANES


@functools.partial(
    pl.kernel,
    mesh=plsc.VectorSubcoreMesh(core_axis_name="c", subcore_axis_name="s"),
    compiler_params=pltpu.CompilerParams(use_tc_tiling_on_sc=False),
    out_type=[
        jax.ShapeDtypeStruct((B, DL), jnp.float32),
        jax.ShapeDtypeStruct((B, DL), jnp.float32),
    ],
    scratch_types=[
        pltpu.VMEM((BPW,), jnp.int32),
        pltpu.VMEM((BPW,), jnp.int32),
        pltpu.VMEM((2, KPW, CHUNK), jnp.int32),
        pltpu.VMEM((2, KPW, CHUNK), jnp.int32),
        pltpu.VMEM((BPW, LANES), jnp.float32),
        pltpu.VMEM((BPW, LANES), jnp.float32),
        pltpu.VMEM((BPW, LANES), jnp.float32),
        pltpu.VMEM((BPW, LANES), jnp.float32),
        pltpu.SemaphoreType.DMA,
        pltpu.SemaphoreType.DMA,
    ],
)
def _sc_gather(uid_hbm, iid_hbm, utab_hbm, itab_hbm, uout_hbm, iout_hbm,
               uids_v, iids_v, uridx_v, iridx_v,
               ustage0_v, ustage1_v, istage0_v, istage1_v, sem_u, sem_i):
    ustage = (ustage0_v, ustage1_v)
    istage = (istage0_v, istage1_v)
    wid = lax.axis_index("s") * NC + lax.axis_index("c")
    b0 = wid * BPW
    pltpu.sync_copy(uid_hbm.at[pl.ds(b0, BPW)], uids_v)
    pltpu.sync_copy(iid_hbm.at[pl.ds(b0, BPW)], iids_v)

    def build_and_fire(d, par):
        for ids_v, ridx_v in ((uids_v, uridx_v), (iids_v, iridx_v)):
            for j in range(BPW // LANES):
                k, c = divmod(j * LANES, CHUNK)
                e = ids_v[pl.ds(j * LANES, LANES)] + d * V
                ridx_v[par, k, pl.ds(c, LANES)] = lax.shift_right_logical(
                    e, 4)
        for k in range(KPW):
            sl = pl.ds(k * CHUNK, CHUNK)
            pltpu.async_copy(utab_hbm.at[uridx_v.at[par, k]],
                             ustage[par].at[sl, :], sem_u)
            pltpu.async_copy(itab_hbm.at[iridx_v.at[par, k]],
                             istage[par].at[sl, :], sem_i)

    def drain(par):
        pltpu.make_async_copy(utab_hbm.at[uridx_v.at[par, 0]],
                              ustage[par], sem_u).wait()
        pltpu.make_async_copy(itab_hbm.at[iridx_v.at[par, 0]],
                              istage[par], sem_i).wait()

    def flush(d, par):
        return
        dst = pl.ds(b0, BPW)
        pltpu.sync_copy(ustage[par],
                        uout_hbm.at[dst, pl.ds(d * LANES, LANES)])
        pltpu.sync_copy(istage[par],
                        iout_hbm.at[dst, pl.ds(d * LANES, LANES)])

    build_and_fire(0, 0)

    def body(t, carry):
        d0 = 2 * t
        build_and_fire(d0 + 1, 1)
        drain(0)
        flush(d0, 0)
        build_and_fire(d0 + 2, 0)
        drain(1)
        flush(d0 + 1, 1)
        return carry

    lax.fori_loop(0, D // 2 - 1, body, 0)
    build_and_fire(D - 1, 1)
    drain(0)
    flush(D - 2, 0)
    drain(1)
    flush(D - 1, 1)


BLK = 1024


def _mlp_body(ustg_ref, istg_ref, uid_ref, iid_ref, w1u_ref, w1i_ref,
              b1_ref, w2_ref, b2_ref, w3_ref, b3_ref, o_ref):
    p = lax.broadcasted_iota(jnp.int32, (1, DL), 1)
    dp = lax.shift_right_logical(p, 4)
    tp = lax.bitwise_and(p, 15)

    def masked(stg_ref, id_ref):
        lane = lax.bitwise_and(dp * V + id_ref[...], 15)
        return jnp.where(lane == tp, stg_ref[...], 0.0)

    h = jnp.dot(masked(ustg_ref, uid_ref), w1u_ref[...],
                preferred_element_type=jnp.float32)
    h += jnp.dot(masked(istg_ref, iid_ref), w1i_ref[...],
                 preferred_element_type=jnp.float32)
    h = jnp.maximum(h + b1_ref[...], 0.0)
    h = jnp.maximum(jnp.dot(h, w2_ref[...],
                            preferred_element_type=jnp.float32)
                    + b2_ref[...], 0.0)
    o = jnp.sum(h * w3_ref[...], axis=1, keepdims=True) + b3_ref[...]
    o_ref[...] = o


def _mlp(ustg, istg, uid2, iid2, w1u, w1i, b1, w2, b2, w3t, b3):
    return pl.pallas_call(
        _mlp_body,
        grid=(B // BLK,),
        in_specs=[
            pl.BlockSpec((BLK, DL), lambda i: (i, 0)),
            pl.BlockSpec((BLK, DL), lambda i: (i, 0)),
            pl.BlockSpec((BLK, 1), lambda i: (i, 0)),
            pl.BlockSpec((BLK, 1), lambda i: (i, 0)),
            pl.BlockSpec((DL, 256), lambda i: (0, 0)),
            pl.BlockSpec((DL, 256), lambda i: (0, 0)),
            pl.BlockSpec((1, 256), lambda i: (0, 0)),
            pl.BlockSpec((256, 64), lambda i: (0, 0)),
            pl.BlockSpec((1, 64), lambda i: (0, 0)),
            pl.BlockSpec((1, 64), lambda i: (0, 0)),
            pl.BlockSpec((1, 1), lambda i: (0, 0)),
        ],
        out_specs=pl.BlockSpec((BLK, 1), lambda i: (i, 0)),
        out_shape=jax.ShapeDtypeStruct((B, 1), jnp.float32),
    )(ustg, istg, uid2, iid2, w1u, w1i, b1, w2, b2, w3t, b3)


def kernel(user_id, item_id, user_table, item_table, W1, b1, W2, b2, W3, b3):
    uid = user_id.astype(jnp.int32)
    iid = item_id.astype(jnp.int32)
    urows16 = user_table.T.reshape(R16, LANES)
    irows16 = item_table.T.reshape(R16, LANES)
    ustg, istg = _sc_gather(uid, iid, urows16, irows16)
    w1uexp = jnp.repeat(W1[:D, :], LANES, axis=0)
    w1iexp = jnp.repeat(W1[D:, :], LANES, axis=0)
    return _mlp(ustg, istg, uid.reshape(B, 1), iid.reshape(B, 1),
                w1uexp, w1iexp, b1.reshape(1, 256),
                W2, b2.reshape(1, 64), W3.reshape(1, 64), b3.reshape(1, 1))


# speedup vs baseline: 4.3961x; 4.3792x over previous
import functools

import jax
import jax.numpy as jnp
from jax import lax
from jax.experimental import pallas as pl
from jax.experimental.pallas import tpu as pltpu
from jax.experimental.pallas import tpu_sc as plsc

B = 16384
D = 32
CHUNK = 128
_info = plsc.get_sparse_core_info()
NC, NS = _info.num_cores, _info.num_subcores
NW = NC * NS
BPW = B // NW
KPW = BPW // CHUNK
NCHUNKS = B // CHUNK


@functools.partial(
    pl.kernel,
    mesh=plsc.VectorSubcoreMesh(core_axis_name="c", subcore_axis_name="s"),
    compiler_params=pltpu.CompilerParams(use_tc_tiling_on_sc=False),
    out_type=[
        jax.ShapeDtypeStruct((NCHUNKS, CHUNK, D), jnp.float32),
        jax.ShapeDtypeStruct((NCHUNKS, CHUNK, D), jnp.float32),
    ],
    scratch_types=[
        pltpu.VMEM((KPW, CHUNK), jnp.int32),
        pltpu.VMEM((KPW, CHUNK), jnp.int32),
        pltpu.VMEM((KPW, CHUNK, D), jnp.float32),
        pltpu.VMEM((KPW, CHUNK, D), jnp.float32),
        pltpu.SemaphoreType.DMA,
        pltpu.SemaphoreType.DMA,
    ],
)
def _sc_gather(uid_hbm, iid_hbm, utab_hbm, itab_hbm, uout_hbm, iout_hbm,
               uidx_v, iidx_v, urows_v, irows_v, sem_u, sem_i):
    wid = lax.axis_index("s") * NC + lax.axis_index("c")
    c0 = wid * KPW
    pltpu.sync_copy(uid_hbm.at[pl.ds(c0, KPW)], uidx_v)
    pltpu.sync_copy(iid_hbm.at[pl.ds(c0, KPW)], iidx_v)
    copies = []
    for k in range(KPW):
        copies.append(pltpu.async_copy(utab_hbm.at[uidx_v.at[k]],
                                       urows_v.at[k], sem_u))
        copies.append(pltpu.async_copy(itab_hbm.at[iidx_v.at[k]],
                                       irows_v.at[k], sem_i))
    for c in copies:
        c.wait()
    pltpu.sync_copy(urows_v, uout_hbm.at[pl.ds(c0, KPW)])
    pltpu.sync_copy(irows_v, iout_hbm.at[pl.ds(c0, KPW)])


V = 1_000_001
TBLK = 8192


def _tr_body(src_ref, o_ref):
    o_ref[...] = src_ref[...].T


def _transpose(tab_t):
    return pl.pallas_call(
        _tr_body,
        grid=(pl.cdiv(V, TBLK),),
        in_specs=[pl.BlockSpec((D, TBLK), lambda i: (0, i))],
        out_specs=pl.BlockSpec((TBLK, D), lambda i: (i, 0)),
        out_shape=jax.ShapeDtypeStruct((V, D), jnp.float32),
    )(tab_t)


BLK = 2048


def _mlp_body(xu_ref, xi_ref, w1u_ref, w1i_ref, b1_ref, w2_ref, b2_ref,
              w3_ref, b3_ref, o_ref):
    h = jnp.dot(xu_ref[...], w1u_ref[...], preferred_element_type=jnp.float32)
    h += jnp.dot(xi_ref[...], w1i_ref[...], preferred_element_type=jnp.float32)
    h = jnp.maximum(h + b1_ref[...], 0.0)
    h = jnp.dot(h, w2_ref[...], preferred_element_type=jnp.float32)
    h = jnp.maximum(h + b2_ref[...], 0.0)
    o = jnp.sum(h * w3_ref[...], axis=1, keepdims=True) + b3_ref[...]
    o_ref[...] = o


def _mlp(xu, xi, w1u, w1i, b1, w2, b2, w3t, b3):
    grid = (B // BLK,)
    return pl.pallas_call(
        _mlp_body,
        grid=grid,
        in_specs=[
            pl.BlockSpec((BLK, D), lambda i: (i, 0)),
            pl.BlockSpec((BLK, D), lambda i: (i, 0)),
            pl.BlockSpec((D, 256), lambda i: (0, 0)),
            pl.BlockSpec((D, 256), lambda i: (0, 0)),
            pl.BlockSpec((1, 256), lambda i: (0, 0)),
            pl.BlockSpec((256, 64), lambda i: (0, 0)),
            pl.BlockSpec((1, 64), lambda i: (0, 0)),
            pl.BlockSpec((1, 64), lambda i: (0, 0)),
            pl.BlockSpec((1, 1), lambda i: (0, 0)),
        ],
        out_specs=pl.BlockSpec((BLK, 1), lambda i: (i, 0)),
        out_shape=jax.ShapeDtypeStruct((B, 1), jnp.float32),
    )(xu, xi, w1u, w1i, b1, w2, b2, w3t, b3)


def kernel(user_id, item_id, user_table, item_table, W1, b1, W2, b2, W3, b3):
    uid = user_id.astype(jnp.int32).reshape(NCHUNKS, CHUNK)
    iid = item_id.astype(jnp.int32).reshape(NCHUNKS, CHUNK)
    utab = _transpose(user_table.T)
    itab = _transpose(item_table.T)
    uemb, iemb = _sc_gather(uid, iid, utab, itab)
    xu = uemb.reshape(B, D)
    xi = iemb.reshape(B, D)
    return _mlp(xu, xi, W1[:D, :], W1[D:, :], b1.reshape(1, 256),
                W2, b2.reshape(1, 64), W3.reshape(1, 64), b3.reshape(1, 1))


# speedup vs baseline: 5.6691x; 1.2896x over previous
import functools

import jax
import jax.numpy as jnp
from jax import lax
from jax.experimental import pallas as pl
from jax.experimental.pallas import tpu as pltpu
from jax.experimental.pallas import tpu_sc as plsc

B = 16384
D = 32
CHUNK = 128
_info = plsc.get_sparse_core_info()
NC, NS = _info.num_cores, _info.num_subcores
NW = NC * NS
BPW = B // NW
KPW = BPW // CHUNK
NCHUNKS = B // CHUNK


@functools.partial(
    pl.kernel,
    mesh=plsc.VectorSubcoreMesh(core_axis_name="c", subcore_axis_name="s"),
    compiler_params=pltpu.CompilerParams(use_tc_tiling_on_sc=False),
    out_type=[
        jax.ShapeDtypeStruct((NCHUNKS, CHUNK, D), jnp.float32),
        jax.ShapeDtypeStruct((NCHUNKS, CHUNK, D), jnp.float32),
    ],
    scratch_types=[
        pltpu.VMEM((KPW, CHUNK), jnp.int32),
        pltpu.VMEM((KPW, CHUNK), jnp.int32),
        pltpu.VMEM((KPW, CHUNK, D), jnp.float32),
        pltpu.VMEM((KPW, CHUNK, D), jnp.float32),
        pltpu.SemaphoreType.DMA,
        pltpu.SemaphoreType.DMA,
    ],
)
def _sc_gather(uid_hbm, iid_hbm, utab_hbm, itab_hbm, uout_hbm, iout_hbm,
               uidx_v, iidx_v, urows_v, irows_v, sem_u, sem_i):
    wid = lax.axis_index("s") * NC + lax.axis_index("c")
    c0 = wid * KPW
    pltpu.sync_copy(uid_hbm.at[pl.ds(c0, KPW)], uidx_v)
    pltpu.sync_copy(iid_hbm.at[pl.ds(c0, KPW)], iidx_v)
    copies = []
    for k in range(KPW):
        copies.append(pltpu.async_copy(utab_hbm.at[uidx_v.at[k]],
                                       urows_v.at[k], sem_u))
        copies.append(pltpu.async_copy(itab_hbm.at[iidx_v.at[k]],
                                       irows_v.at[k], sem_i))
    for c in copies:
        c.wait()
    pltpu.sync_copy(urows_v, uout_hbm.at[pl.ds(c0, KPW)])
    pltpu.sync_copy(irows_v, iout_hbm.at[pl.ds(c0, KPW)])


BLK = 2048


def _mlp_body(xu_ref, xi_ref, w1u_ref, w1i_ref, b1_ref, w2_ref, b2_ref,
              w3_ref, b3_ref, o_ref):
    h = jnp.dot(xu_ref[...], w1u_ref[...], preferred_element_type=jnp.float32)
    h += jnp.dot(xi_ref[...], w1i_ref[...], preferred_element_type=jnp.float32)
    h = jnp.maximum(h + b1_ref[...], 0.0)
    h = jnp.dot(h, w2_ref[...], preferred_element_type=jnp.float32)
    h = jnp.maximum(h + b2_ref[...], 0.0)
    o = jnp.sum(h * w3_ref[...], axis=1, keepdims=True) + b3_ref[...]
    o_ref[...] = o


def _mlp(xu, xi, w1u, w1i, b1, w2, b2, w3t, b3):
    grid = (B // BLK,)
    return pl.pallas_call(
        _mlp_body,
        grid=grid,
        in_specs=[
            pl.BlockSpec((BLK, D), lambda i: (i, 0)),
            pl.BlockSpec((BLK, D), lambda i: (i, 0)),
            pl.BlockSpec((D, 256), lambda i: (0, 0)),
            pl.BlockSpec((D, 256), lambda i: (0, 0)),
            pl.BlockSpec((1, 256), lambda i: (0, 0)),
            pl.BlockSpec((256, 64), lambda i: (0, 0)),
            pl.BlockSpec((1, 64), lambda i: (0, 0)),
            pl.BlockSpec((1, 64), lambda i: (0, 0)),
            pl.BlockSpec((1, 1), lambda i: (0, 0)),
        ],
        out_specs=pl.BlockSpec((BLK, 1), lambda i: (i, 0)),
        out_shape=jax.ShapeDtypeStruct((B, 1), jnp.float32),
    )(xu, xi, w1u, w1i, b1, w2, b2, w3t, b3)


def kernel(user_id, item_id, user_table, item_table, W1, b1, W2, b2, W3, b3):
    uid = user_id.astype(jnp.int32).reshape(NCHUNKS, CHUNK)
    iid = item_id.astype(jnp.int32).reshape(NCHUNKS, CHUNK)
    uemb, iemb = _sc_gather(uid, iid, user_table, item_table)
    xu = uemb.reshape(B, D)
    xi = iemb.reshape(B, D)
    return _mlp(xu, xi, W1[:D, :], W1[D:, :], b1.reshape(1, 256),
                W2, b2.reshape(1, 64), W3.reshape(1, 64), b3.reshape(1, 1))
